# Initial kernel scaffold; baseline (speedup 1.0000x reference)
import jax
import jax.numpy as jnp
from jax.experimental import pallas as pl


def kernel(x, edge_index, Wl1, Wr1, b1, Wl2, Wr2, b2):
    raise NotImplementedError("write your pallas kernel here")


# speedup vs baseline: 5.0710x; 5.0710x over previous
import functools

import jax
import jax.numpy as jnp
from jax import lax
from jax.experimental import pallas as pl
from jax.experimental.pallas import tpu as pltpu
from jax.experimental.pallas import tpu_sc as plsc

N = 10000
E = 320000
D = 128
DH = 64
DA = 80
NC = 2
NS = 16
EPT = E // NS
G = 80
NCH = EPT // G
RPS = N // NS
ZB = 125

BLK = 2000


def _sc_segment_sum(p_split, src_t, dst_t):
    mesh = plsc.VectorSubcoreMesh(core_axis_name="c", subcore_axis_name="s")

    @functools.partial(
        pl.kernel,
        mesh=mesh,
        compiler_params=pltpu.CompilerParams(use_tc_tiling_on_sc=False),
        out_type=jax.ShapeDtypeStruct((NC, N, DA), jnp.float32),
        scratch_types=[
            pltpu.VMEM((NCH, G), jnp.int32),
            pltpu.VMEM((NCH, G), jnp.int32),
            pltpu.VMEM((G, DA), jnp.float32),
            pltpu.VMEM((ZB, DA), jnp.float32),
            pltpu.VMEM_SHARED((N, DA), jnp.float32),
        ],
    )
    def k(p_hbm, src_hbm, dst_hbm, acc_hbm, src_v, dst_v, rows_v, buf_v, acc_sh):
        c = lax.axis_index("c")
        s = lax.axis_index("s")

        pltpu.sync_copy(src_hbm.at[s], src_v)
        pltpu.sync_copy(dst_hbm.at[s], dst_v)

        cvec = jnp.full((16,), c, dtype=jnp.int32)

        def addbase(i, carry):
            for j in range(G // 16):
                sl = (i, pl.ds(j * 16, 16))
                src_v[sl] = src_v[sl] * 2 + cvec
            return carry

        lax.fori_loop(0, NCH, addbase, 0)

        zeros16 = jnp.zeros((16,), jnp.float32)

        def zrow(i, carry):
            for j in range(DA // 16):
                buf_v[i, pl.ds(j * 16, 16)] = zeros16
            return carry

        lax.fori_loop(0, ZB, zrow, 0)

        def zslab(i, carry):
            pltpu.sync_copy(buf_v, acc_sh.at[pl.ds(s * RPS + i * ZB, ZB)])
            return carry

        lax.fori_loop(0, RPS // ZB, zslab, 0)
        plsc.subcore_barrier()

        def chunk(j, carry):
            pltpu.sync_copy(p_hbm.at[src_v.at[j]], rows_v)
            pltpu.sync_copy(rows_v, acc_sh.at[dst_v.at[j]], add=True)
            return carry

        lax.fori_loop(0, NCH, chunk, 0)
        plsc.subcore_barrier()

        def wslab(i, carry):
            pltpu.sync_copy(acc_sh.at[pl.ds(s * RPS + i * ZB, ZB)], buf_v)
            pltpu.sync_copy(buf_v, acc_hbm.at[c, pl.ds(s * RPS + i * ZB, ZB)])
            return carry

        lax.fori_loop(0, RPS // ZB, wslab, 0)

    return k(p_split, src_t, dst_t)


def _aug_halves(p, out_ref):
    out_ref[:, 0, 0:DH] = p[:, 0:DH]
    out_ref[:, 1, 0:DH] = p[:, DH:D]
    col = lax.broadcasted_iota(jnp.int32, (BLK, DA - DH), 1)
    ones_pad = jnp.where(col == 0, 1.0, 0.0).astype(jnp.float32)
    out_ref[:, 0, DH:DA] = ones_pad
    out_ref[:, 1, DH:DA] = ones_pad


def _tc_project_aug(x, Wl):
    def body(x_ref, wl_ref, out_ref):
        p = jnp.dot(x_ref[...], wl_ref[...], preferred_element_type=jnp.float32)
        _aug_halves(p, out_ref)

    return pl.pallas_call(
        body,
        grid=(N // BLK,),
        in_specs=[
            pl.BlockSpec((BLK, D), lambda i: (i, 0)),
            pl.BlockSpec((D, D), lambda i: (0, 0)),
        ],
        out_specs=pl.BlockSpec((BLK, 2, DA), lambda i: (i, 0, 0)),
        out_shape=jax.ShapeDtypeStruct((N, 2, DA), jnp.float32),
    )(x, Wl)


def _agg_from_acc(acc_ref):
    a0 = acc_ref[0]
    a1 = acc_ref[1]
    deg = jnp.clip(a0[:, DH:DH + 1], 1.0, None)
    agg = jnp.concatenate([a0[:, 0:DH], a1[:, 0:DH]], axis=1) / deg
    return agg


def _tc_mid(acc, x, Wr1, b1, Wl2, Wr2, b2):
    def body(acc_ref, x_ref, wr1_ref, b1_ref, wl2_ref, wr2_ref, b2_ref,
             p2_ref, r2_ref):
        agg = _agg_from_acc(acc_ref)
        h = jnp.maximum(
            agg + jnp.dot(x_ref[...], wr1_ref[...],
                          preferred_element_type=jnp.float32) + b1_ref[...],
            0.0)
        p2 = jnp.dot(h, wl2_ref[...], preferred_element_type=jnp.float32)
        _aug_halves(p2, p2_ref)
        r2_ref[...] = jnp.dot(h, wr2_ref[...],
                              preferred_element_type=jnp.float32) + b2_ref[...]

    return pl.pallas_call(
        body,
        grid=(N // BLK,),
        in_specs=[
            pl.BlockSpec((NC, BLK, DA), lambda i: (0, i, 0)),
            pl.BlockSpec((BLK, D), lambda i: (i, 0)),
            pl.BlockSpec((D, D), lambda i: (0, 0)),
            pl.BlockSpec((1, D), lambda i: (0, 0)),
            pl.BlockSpec((D, D), lambda i: (0, 0)),
            pl.BlockSpec((D, D), lambda i: (0, 0)),
            pl.BlockSpec((1, D), lambda i: (0, 0)),
        ],
        out_specs=[
            pl.BlockSpec((BLK, 2, DA), lambda i: (i, 0, 0)),
            pl.BlockSpec((BLK, D), lambda i: (i, 0)),
        ],
        out_shape=[
            jax.ShapeDtypeStruct((N, 2, DA), jnp.float32),
            jax.ShapeDtypeStruct((N, D), jnp.float32),
        ],
    )(acc, x, Wr1, b1, Wl2, Wr2, b2)


def _tc_final(acc, r2):
    def body(acc_ref, r2_ref, out_ref):
        t = _agg_from_acc(acc_ref) + r2_ref[...]
        m = jnp.max(t, axis=-1, keepdims=True)
        lse = m + jnp.log(jnp.sum(jnp.exp(t - m), axis=-1, keepdims=True))
        out_ref[...] = t - lse

    return pl.pallas_call(
        body,
        grid=(N // BLK,),
        in_specs=[
            pl.BlockSpec((NC, BLK, DA), lambda i: (0, i, 0)),
            pl.BlockSpec((BLK, D), lambda i: (i, 0)),
        ],
        out_specs=pl.BlockSpec((BLK, D), lambda i: (i, 0)),
        out_shape=jax.ShapeDtypeStruct((N, D), jnp.float32),
    )(acc, r2)


def kernel(x, edge_index, Wl1, Wr1, b1, Wl2, Wr2, b2):
    src = edge_index[0].astype(jnp.int32).reshape(NS, NCH, G)
    dst = edge_index[1].astype(jnp.int32).reshape(NS, NCH, G)
    b1r = b1.reshape(1, D)
    b2r = b2.reshape(1, D)

    p1 = _tc_project_aug(x, Wl1).reshape(2 * N, DA)
    acc1 = _sc_segment_sum(p1, src, dst)
    p2, r2 = _tc_mid(acc1, x, Wr1, b1r, Wl2, Wr2, b2r)
    acc2 = _sc_segment_sum(p2.reshape(2 * N, DA), src, dst)
    return _tc_final(acc2, r2)


# speedup vs baseline: 8.1737x; 1.6118x over previous
import functools

import jax
import jax.numpy as jnp
from jax import lax
from jax.experimental import pallas as pl
from jax.experimental.pallas import tpu as pltpu
from jax.experimental.pallas import tpu_sc as plsc

N = 10000
E = 320000
D = 128
DH = 64
DA = 80
NC = 2
NS = 16
EPT = E // NS
G = 80
NCH = EPT // G
RPS = N // NS
ZB = 125

BLK = 2000


def _sc_segment_sum(p_split, src_t, dst_t):
    mesh = plsc.VectorSubcoreMesh(core_axis_name="c", subcore_axis_name="s")

    @functools.partial(
        pl.kernel,
        mesh=mesh,
        compiler_params=pltpu.CompilerParams(use_tc_tiling_on_sc=False),
        out_type=jax.ShapeDtypeStruct((NC, N, DA), jnp.float32),
        scratch_types=[
            pltpu.VMEM((NCH, G), jnp.int32),
            pltpu.VMEM((NCH, G), jnp.int32),
            pltpu.VMEM((G, DA), jnp.float32),
            pltpu.VMEM((G, DA), jnp.float32),
            pltpu.VMEM((ZB, DA), jnp.float32),
            pltpu.VMEM_SHARED((N, DA), jnp.float32),
            pltpu.SemaphoreType.DMA,
            pltpu.SemaphoreType.DMA,
        ],
    )
    def k(p_hbm, src_hbm, dst_hbm, acc_hbm, src_v, dst_v, rows0, rows1,
          buf_v, acc_sh, sem0, sem1):
        c = lax.axis_index("c")
        s = lax.axis_index("s")

        pltpu.sync_copy(src_hbm.at[s], src_v)
        pltpu.sync_copy(dst_hbm.at[s], dst_v)

        cvec = jnp.full((16,), c, dtype=jnp.int32)

        def addbase(i, carry):
            for j in range(G // 16):
                sl = (i, pl.ds(j * 16, 16))
                src_v[sl] = src_v[sl] * 2 + cvec
            return carry

        lax.fori_loop(0, NCH, addbase, 0)

        zeros16 = jnp.zeros((16,), jnp.float32)

        def zrow(i, carry):
            for j in range(DA // 16):
                buf_v[i, pl.ds(j * 16, 16)] = zeros16
            return carry

        lax.fori_loop(0, ZB, zrow, 0)

        def zslab(i, carry):
            pltpu.sync_copy(buf_v, acc_sh.at[pl.ds(s * RPS + i * ZB, ZB)])
            return carry

        lax.fori_loop(0, RPS // ZB, zslab, 0)
        plsc.subcore_barrier()

        def start_g(j, buf, sem):
            pltpu.async_copy(p_hbm.at[src_v.at[j]], buf, sem)

        def wait_g(j, buf, sem):
            pltpu.make_async_copy(p_hbm.at[src_v.at[j]], buf, sem).wait()

        def start_s(j, buf, sem):
            pltpu.async_copy(buf, acc_sh.at[dst_v.at[j]], sem, add=True)

        def wait_s(j, buf, sem):
            pltpu.make_async_copy(buf, acc_sh.at[dst_v.at[j]], sem).wait()

        def step(j, buf, sem, prefetch):
            wait_g(j, buf, sem)
            start_s(j, buf, sem)
            wait_s(j, buf, sem)
            if prefetch:
                start_g(j + 2, buf, sem)

        start_g(0, rows0, sem0)
        start_g(1, rows1, sem1)

        def pair(t, carry):
            step(2 * t, rows0, sem0, True)
            step(2 * t + 1, rows1, sem1, True)
            return carry

        lax.fori_loop(0, (NCH - 2) // 2, pair, 0)
        step(NCH - 2, rows0, sem0, False)
        step(NCH - 1, rows1, sem1, False)
        plsc.subcore_barrier()

        def wslab(i, carry):
            pltpu.sync_copy(acc_sh.at[pl.ds(s * RPS + i * ZB, ZB)], buf_v)
            pltpu.sync_copy(buf_v, acc_hbm.at[c, pl.ds(s * RPS + i * ZB, ZB)])
            return carry

        lax.fori_loop(0, RPS // ZB, wslab, 0)

    return k(p_split, src_t, dst_t)


def _aug_halves(p, out_ref):
    out_ref[:, 0, 0:DH] = p[:, 0:DH]
    out_ref[:, 1, 0:DH] = p[:, DH:D]
    col = lax.broadcasted_iota(jnp.int32, (BLK, DA - DH), 1)
    ones_pad = jnp.where(col == 0, 1.0, 0.0).astype(jnp.float32)
    out_ref[:, 0, DH:DA] = ones_pad
    out_ref[:, 1, DH:DA] = ones_pad


def _tc_project_aug(x, Wl):
    def body(x_ref, wl_ref, out_ref):
        p = jnp.dot(x_ref[...], wl_ref[...], preferred_element_type=jnp.float32)
        _aug_halves(p, out_ref)

    return pl.pallas_call(
        body,
        grid=(N // BLK,),
        in_specs=[
            pl.BlockSpec((BLK, D), lambda i: (i, 0)),
            pl.BlockSpec((D, D), lambda i: (0, 0)),
        ],
        out_specs=pl.BlockSpec((BLK, 2, DA), lambda i: (i, 0, 0)),
        out_shape=jax.ShapeDtypeStruct((N, 2, DA), jnp.float32),
    )(x, Wl)


def _agg_from_acc(acc_ref):
    a0 = acc_ref[0]
    a1 = acc_ref[1]
    deg = jnp.clip(a0[:, DH:DH + 1], 1.0, None)
    agg = jnp.concatenate([a0[:, 0:DH], a1[:, 0:DH]], axis=1) / deg
    return agg


def _tc_mid(acc, x, Wr1, b1, Wl2, Wr2, b2):
    def body(acc_ref, x_ref, wr1_ref, b1_ref, wl2_ref, wr2_ref, b2_ref,
             p2_ref, r2_ref):
        agg = _agg_from_acc(acc_ref)
        h = jnp.maximum(
            agg + jnp.dot(x_ref[...], wr1_ref[...],
                          preferred_element_type=jnp.float32) + b1_ref[...],
            0.0)
        p2 = jnp.dot(h, wl2_ref[...], preferred_element_type=jnp.float32)
        _aug_halves(p2, p2_ref)
        r2_ref[...] = jnp.dot(h, wr2_ref[...],
                              preferred_element_type=jnp.float32) + b2_ref[...]

    return pl.pallas_call(
        body,
        grid=(N // BLK,),
        in_specs=[
            pl.BlockSpec((NC, BLK, DA), lambda i: (0, i, 0)),
            pl.BlockSpec((BLK, D), lambda i: (i, 0)),
            pl.BlockSpec((D, D), lambda i: (0, 0)),
            pl.BlockSpec((1, D), lambda i: (0, 0)),
            pl.BlockSpec((D, D), lambda i: (0, 0)),
            pl.BlockSpec((D, D), lambda i: (0, 0)),
            pl.BlockSpec((1, D), lambda i: (0, 0)),
        ],
        out_specs=[
            pl.BlockSpec((BLK, 2, DA), lambda i: (i, 0, 0)),
            pl.BlockSpec((BLK, D), lambda i: (i, 0)),
        ],
        out_shape=[
            jax.ShapeDtypeStruct((N, 2, DA), jnp.float32),
            jax.ShapeDtypeStruct((N, D), jnp.float32),
        ],
    )(acc, x, Wr1, b1, Wl2, Wr2, b2)


def _tc_final(acc, r2):
    def body(acc_ref, r2_ref, out_ref):
        t = _agg_from_acc(acc_ref) + r2_ref[...]
        m = jnp.max(t, axis=-1, keepdims=True)
        lse = m + jnp.log(jnp.sum(jnp.exp(t - m), axis=-1, keepdims=True))
        out_ref[...] = t - lse

    return pl.pallas_call(
        body,
        grid=(N // BLK,),
        in_specs=[
            pl.BlockSpec((NC, BLK, DA), lambda i: (0, i, 0)),
            pl.BlockSpec((BLK, D), lambda i: (i, 0)),
        ],
        out_specs=pl.BlockSpec((BLK, D), lambda i: (i, 0)),
        out_shape=jax.ShapeDtypeStruct((N, D), jnp.float32),
    )(acc, r2)


def kernel(x, edge_index, Wl1, Wr1, b1, Wl2, Wr2, b2):
    src = edge_index[0].astype(jnp.int32).reshape(NS, NCH, G)
    dst = edge_index[1].astype(jnp.int32).reshape(NS, NCH, G)
    b1r = b1.reshape(1, D)
    b2r = b2.reshape(1, D)

    p1 = _tc_project_aug(x, Wl1).reshape(2 * N, DA)
    acc1 = _sc_segment_sum(p1, src, dst)
    p2, r2 = _tc_mid(acc1, x, Wr1, b1r, Wl2, Wr2, b2r)
    acc2 = _sc_segment_sum(p2.reshape(2 * N, DA), src, dst)
    return _tc_final(acc2, r2)


# speedup vs baseline: 9.0595x; 1.1084x over previous
import functools

import jax
import jax.numpy as jnp
from jax import lax
from jax.experimental import pallas as pl
from jax.experimental.pallas import tpu as pltpu
from jax.experimental.pallas import tpu_sc as plsc

N = 10000
E = 320000
D = 128
DH = 64
NC = 2
NS = 16
EPT = E // NS
G = 80
NCH = EPT // G
RPS = N // NS
ZB = 125
ND = 10240
DPS = ND // NS
DROWS = NCH // NC

BLK = 2000


def _sc_segment_sum(p_split, src_t, dst_t, compute_deg):
    mesh = plsc.VectorSubcoreMesh(core_axis_name="c", subcore_axis_name="s")

    out_type = [jax.ShapeDtypeStruct((NC, N, DH), jnp.float32)]
    scratch = [
        pltpu.VMEM((NCH, G), jnp.int32),
        pltpu.VMEM((NCH, G), jnp.int32),
        pltpu.VMEM((G, DH), jnp.float32),
        pltpu.VMEM((G, DH), jnp.float32),
        pltpu.VMEM((ZB, DH), jnp.float32),
        pltpu.VMEM_SHARED((N, DH), jnp.float32),
        pltpu.SemaphoreType.DMA,
        pltpu.SemaphoreType.DMA,
    ]
    if compute_deg:
        out_type.append(jax.ShapeDtypeStruct((NC, ND), jnp.float32))
        scratch += [
            pltpu.VMEM((ND,), jnp.float32),
            pltpu.VMEM((DPS,), jnp.float32),
            pltpu.VMEM_SHARED((NS, ND), jnp.float32),
        ]

    @functools.partial(
        pl.kernel,
        mesh=mesh,
        compiler_params=pltpu.CompilerParams(
            use_tc_tiling_on_sc=False, needs_layout_passes=False),
        out_type=tuple(out_type),
        scratch_types=scratch,
    )
    def k(p_hbm, src_hbm, dst_hbm, *refs):
        if compute_deg:
            (acc_hbm, deg_hbm, src_v, dst_v, rows0, rows1, buf_v, acc_sh,
             sem0, sem1, degv, dsum, dstage) = refs
        else:
            (acc_hbm, src_v, dst_v, rows0, rows1, buf_v, acc_sh,
             sem0, sem1) = refs

        c = lax.axis_index("c")
        s = lax.axis_index("s")

        pltpu.sync_copy(src_hbm.at[s], src_v)
        pltpu.sync_copy(dst_hbm.at[s], dst_v)

        cvec = jnp.full((16,), c, dtype=jnp.int32)

        def addbase(i, carry):
            for j in range(G // 16):
                sl = (i, pl.ds(j * 16, 16))
                src_v[sl] = src_v[sl] * 2 + cvec
            return carry

        lax.fori_loop(0, NCH, addbase, 0)

        zeros16 = jnp.zeros((16,), jnp.float32)

        def zrow(i, carry):
            for j in range(DH // 16):
                buf_v[i, pl.ds(j * 16, 16)] = zeros16
            return carry

        lax.fori_loop(0, ZB, zrow, 0)

        def zslab(i, carry):
            pltpu.sync_copy(buf_v, acc_sh.at[pl.ds(s * RPS + i * ZB, ZB)])
            return carry

        lax.fori_loop(0, RPS // ZB, zslab, 0)

        if compute_deg:
            def zdeg(i, carry):
                degv[pl.ds(i * 16, 16)] = zeros16
                return carry

            lax.fori_loop(0, ND // 16, zdeg, 0)

        plsc.subcore_barrier()

        def start_g(j, buf, sem):
            pltpu.async_copy(p_hbm.at[src_v.at[j]], buf, sem)

        def wait_g(j, buf, sem):
            pltpu.make_async_copy(p_hbm.at[src_v.at[j]], buf, sem).wait()

        def start_s(j, buf, sem):
            pltpu.async_copy(buf, acc_sh.at[dst_v.at[j]], sem, add=True)

        def wait_s(j, buf, sem):
            pltpu.make_async_copy(buf, acc_sh.at[dst_v.at[j]], sem).wait()

        ones16 = jnp.ones((16,), jnp.float32)

        def deg_row(t):
            row = c * DROWS + t
            for j in range(G // 16):
                idx = dst_v[row, pl.ds(j * 16, 16)]
                plsc.addupdate_scatter(degv, [idx], ones16)

        def step(j, buf, sem, prefetch):
            wait_g(j, buf, sem)
            start_s(j, buf, sem)
            wait_s(j, buf, sem)
            if prefetch:
                start_g(j + 2, buf, sem)

        start_g(0, rows0, sem0)
        start_g(1, rows1, sem1)

        def pair(t, carry):
            wait_g(2 * t, rows0, sem0)
            start_s(2 * t, rows0, sem0)
            if compute_deg:
                deg_row(t)
            wait_s(2 * t, rows0, sem0)
            start_g(2 * t + 2, rows0, sem0)
            step(2 * t + 1, rows1, sem1, True)
            return carry

        lax.fori_loop(0, (NCH - 2) // 2, pair, 0)
        step(NCH - 2, rows0, sem0, False)
        step(NCH - 1, rows1, sem1, False)
        if compute_deg:
            deg_row(DROWS - 1)
        plsc.subcore_barrier()

        def wslab(i, carry):
            pltpu.sync_copy(acc_sh.at[pl.ds(s * RPS + i * ZB, ZB)], buf_v)
            pltpu.sync_copy(buf_v, acc_hbm.at[c, pl.ds(s * RPS + i * ZB, ZB)])
            return carry

        lax.fori_loop(0, RPS // ZB, wslab, 0)

        if compute_deg:
            pltpu.sync_copy(degv, dstage.at[s])
            plsc.subcore_barrier()

            def dzero(i, carry):
                dsum[pl.ds(i * 16, 16)] = zeros16
                return carry

            lax.fori_loop(0, DPS // 16, dzero, 0)

            def dred(r, carry):
                pltpu.sync_copy(dstage.at[r, pl.ds(s * DPS, DPS)], degv.at[pl.ds(0, DPS)])
                for i in range(DPS // 16):
                    sl = pl.ds(i * 16, 16)
                    dsum[sl] = dsum[sl] + degv[sl]
                return carry

            lax.fori_loop(0, NS, dred, 0)
            pltpu.sync_copy(dsum, deg_hbm.at[c, pl.ds(s * DPS, DPS)])

    return k(p_split, src_t, dst_t)


def _split_halves(p, out_ref):
    out_ref[:, 0, :] = p[:, 0:DH]
    out_ref[:, 1, :] = p[:, DH:D]


def _tc_project(x, Wl):
    def body(x_ref, wl_ref, out_ref):
        p = jnp.dot(x_ref[...], wl_ref[...], preferred_element_type=jnp.float32)
        _split_halves(p, out_ref)

    return pl.pallas_call(
        body,
        grid=(N // BLK,),
        in_specs=[
            pl.BlockSpec((BLK, D), lambda i: (i, 0)),
            pl.BlockSpec((D, D), lambda i: (0, 0)),
        ],
        out_specs=pl.BlockSpec((BLK, 2, DH), lambda i: (i, 0, 0)),
        out_shape=jax.ShapeDtypeStruct((N, 2, DH), jnp.float32),
    )(x, Wl)


def _agg_from_acc(acc_ref, d0_ref, d1_ref):
    deg = jnp.clip(d0_ref[...] + d1_ref[...], 1.0, None)
    agg = jnp.concatenate([acc_ref[0], acc_ref[1]], axis=1) / deg
    return agg


def _tc_mid(acc, d0, d1, x, Wr1, b1, Wl2, Wr2, b2):
    def body(acc_ref, d0_ref, d1_ref, x_ref, wr1_ref, b1_ref, wl2_ref,
             wr2_ref, b2_ref, p2_ref, r2_ref):
        agg = _agg_from_acc(acc_ref, d0_ref, d1_ref)
        h = jnp.maximum(
            agg + jnp.dot(x_ref[...], wr1_ref[...],
                          preferred_element_type=jnp.float32) + b1_ref[...],
            0.0)
        p2 = jnp.dot(h, wl2_ref[...], preferred_element_type=jnp.float32)
        _split_halves(p2, p2_ref)
        r2_ref[...] = jnp.dot(h, wr2_ref[...],
                              preferred_element_type=jnp.float32) + b2_ref[...]

    return pl.pallas_call(
        body,
        grid=(N // BLK,),
        in_specs=[
            pl.BlockSpec((NC, BLK, DH), lambda i: (0, i, 0)),
            pl.BlockSpec((BLK, 1), lambda i: (i, 0)),
            pl.BlockSpec((BLK, 1), lambda i: (i, 0)),
            pl.BlockSpec((BLK, D), lambda i: (i, 0)),
            pl.BlockSpec((D, D), lambda i: (0, 0)),
            pl.BlockSpec((1, D), lambda i: (0, 0)),
            pl.BlockSpec((D, D), lambda i: (0, 0)),
            pl.BlockSpec((D, D), lambda i: (0, 0)),
            pl.BlockSpec((1, D), lambda i: (0, 0)),
        ],
        out_specs=[
            pl.BlockSpec((BLK, 2, DH), lambda i: (i, 0, 0)),
            pl.BlockSpec((BLK, D), lambda i: (i, 0)),
        ],
        out_shape=[
            jax.ShapeDtypeStruct((N, 2, DH), jnp.float32),
            jax.ShapeDtypeStruct((N, D), jnp.float32),
        ],
    )(acc, d0, d1, x, Wr1, b1, Wl2, Wr2, b2)


def _tc_final(acc, d0, d1, r2):
    def body(acc_ref, d0_ref, d1_ref, r2_ref, out_ref):
        t = _agg_from_acc(acc_ref, d0_ref, d1_ref) + r2_ref[...]
        m = jnp.max(t, axis=-1, keepdims=True)
        lse = m + jnp.log(jnp.sum(jnp.exp(t - m), axis=-1, keepdims=True))
        out_ref[...] = t - lse

    return pl.pallas_call(
        body,
        grid=(N // BLK,),
        in_specs=[
            pl.BlockSpec((NC, BLK, DH), lambda i: (0, i, 0)),
            pl.BlockSpec((BLK, 1), lambda i: (i, 0)),
            pl.BlockSpec((BLK, 1), lambda i: (i, 0)),
            pl.BlockSpec((BLK, D), lambda i: (i, 0)),
        ],
        out_specs=pl.BlockSpec((BLK, D), lambda i: (i, 0)),
        out_shape=jax.ShapeDtypeStruct((N, D), jnp.float32),
    )(acc, d0, d1, r2)


def kernel(x, edge_index, Wl1, Wr1, b1, Wl2, Wr2, b2):
    src = edge_index[0].astype(jnp.int32).reshape(NS, NCH, G)
    dst = edge_index[1].astype(jnp.int32).reshape(NS, NCH, G)
    b1r = b1.reshape(1, D)
    b2r = b2.reshape(1, D)

    p1 = _tc_project(x, Wl1).reshape(2 * N, DH)
    acc1, deg = _sc_segment_sum(p1, src, dst, compute_deg=True)
    d0 = deg[0, :N].reshape(N, 1)
    d1 = deg[1, :N].reshape(N, 1)
    p2, r2 = _tc_mid(acc1, d0, d1, x, Wr1, b1r, Wl2, Wr2, b2r)
    (acc2,) = _sc_segment_sum(p2.reshape(2 * N, DH), src, dst, compute_deg=False)
    return _tc_final(acc2, d0, d1, r2)


# speedup vs baseline: 12.0310x; 1.3280x over previous
import functools

import jax
import jax.numpy as jnp
from jax import lax
from jax.experimental import pallas as pl
from jax.experimental.pallas import tpu as pltpu
from jax.experimental.pallas import tpu_sc as plsc

N = 10000
E = 320000
D = 128
DH = 64
NC = 2
NS = 16
EPT = E // NS
G = 80
NCH = EPT // G
RPS = N // NS
ZB = 125
ND = 10240
DPS = ND // NS
DROWS = NCH // NC

BLK = 2000


NBUF = 4


def _sc_segment_sum(p_split, src_t, dst_t, compute_deg):
    mesh = plsc.VectorSubcoreMesh(core_axis_name="c", subcore_axis_name="s")

    out_type = [jax.ShapeDtypeStruct((NC, N, DH), jnp.float32)]
    scratch = [
        pltpu.VMEM((NCH, G), jnp.int32),
        pltpu.VMEM((NCH, G), jnp.int32),
    ] + [pltpu.VMEM((G, DH), jnp.float32) for _ in range(NBUF)] + [
        pltpu.VMEM((ZB, DH), jnp.float32),
        pltpu.VMEM_SHARED((N, DH), jnp.float32),
    ] + [pltpu.SemaphoreType.DMA for _ in range(NBUF)]
    if compute_deg:
        out_type.append(jax.ShapeDtypeStruct((NC, ND), jnp.float32))
        scratch += [
            pltpu.VMEM((ND,), jnp.float32),
            pltpu.VMEM((DPS,), jnp.float32),
            pltpu.VMEM_SHARED((NS, ND), jnp.float32),
        ]

    @functools.partial(
        pl.kernel,
        mesh=mesh,
        compiler_params=pltpu.CompilerParams(
            use_tc_tiling_on_sc=False, needs_layout_passes=False),
        out_type=tuple(out_type),
        scratch_types=scratch,
    )
    def k(p_hbm, src_hbm, dst_hbm, *refs):
        if compute_deg:
            acc_hbm, deg_hbm = refs[0], refs[1]
            rest = refs[2:]
        else:
            acc_hbm = refs[0]
            rest = refs[1:]
        src_v, dst_v = rest[0], rest[1]
        bufs = rest[2:2 + NBUF]
        buf_v, acc_sh = rest[2 + NBUF], rest[3 + NBUF]
        sems = rest[4 + NBUF:4 + 2 * NBUF]
        if compute_deg:
            degv, dsum, dstage = rest[4 + 2 * NBUF:]

        c = lax.axis_index("c")
        s = lax.axis_index("s")

        pltpu.sync_copy(src_hbm.at[c, s], src_v)
        pltpu.sync_copy(dst_hbm.at[s], dst_v)

        zeros16 = jnp.zeros((16,), jnp.float32)

        def zrow(i, carry):
            for j in range(DH // 16):
                buf_v[i, pl.ds(j * 16, 16)] = zeros16
            return carry

        lax.fori_loop(0, ZB, zrow, 0)

        def zslab(i, carry):
            pltpu.sync_copy(buf_v, acc_sh.at[pl.ds(s * RPS + i * ZB, ZB)])
            return carry

        lax.fori_loop(0, RPS // ZB, zslab, 0)

        if compute_deg:
            def zdeg(i, carry):
                degv[pl.ds(i * 16, 16)] = zeros16
                return carry

            lax.fori_loop(0, ND // 16, zdeg, 0)

        plsc.subcore_barrier()

        def start_g(j, buf, sem):
            pltpu.async_copy(p_hbm.at[src_v.at[j]], buf, sem)

        def wait_g(j, buf, sem):
            pltpu.make_async_copy(p_hbm.at[src_v.at[j]], buf, sem).wait()

        def start_s(j, buf, sem):
            pltpu.async_copy(buf, acc_sh.at[dst_v.at[j]], sem, add=True)

        def wait_s(j, buf, sem):
            pltpu.make_async_copy(buf, acc_sh.at[dst_v.at[j]], sem).wait()

        ones16 = jnp.ones((16,), jnp.float32)

        def deg_row(t):
            row = c * DROWS + t
            for j in range(G // 16):
                idx = dst_v[row, pl.ds(j * 16, 16)]
                plsc.addupdate_scatter(degv, [idx], ones16)

        for k in range(NBUF):
            start_g(k, bufs[k], sems[k])

        QT = (NCH - 2 * NBUF) // NBUF + 1
        JE = QT * NBUF

        def quad(t, carry):
            for k in range(NBUF):
                j = NBUF * t + k
                wait_g(j, bufs[k], sems[k])
                start_s(j, bufs[k], sems[k])
                if compute_deg and k % 2 == 0:
                    deg_row(2 * t + k // 2)
                wait_s(j, bufs[k], sems[k])
                start_g(j + NBUF, bufs[k], sems[k])
            return carry

        lax.fori_loop(0, QT, quad, 0)

        for j in range(JE, NCH):
            k = j % NBUF
            wait_g(j, bufs[k], sems[k])
            start_s(j, bufs[k], sems[k])
            wait_s(j, bufs[k], sems[k])
            if j + NBUF < NCH:
                start_g(j + NBUF, bufs[k], sems[k])
        if compute_deg:
            for r in range(2 * QT, DROWS):
                deg_row(r)
        plsc.subcore_barrier()

        def wslab(i, carry):
            pltpu.sync_copy(acc_sh.at[pl.ds(s * RPS + i * ZB, ZB)], buf_v)
            pltpu.sync_copy(buf_v, acc_hbm.at[c, pl.ds(s * RPS + i * ZB, ZB)])
            return carry

        lax.fori_loop(0, RPS // ZB, wslab, 0)

        if compute_deg:
            pltpu.sync_copy(degv, dstage.at[s])
            plsc.subcore_barrier()

            def dzero(i, carry):
                dsum[pl.ds(i * 16, 16)] = zeros16
                return carry

            lax.fori_loop(0, DPS // 16, dzero, 0)

            def dred(r, carry):
                pltpu.sync_copy(dstage.at[r, pl.ds(s * DPS, DPS)], degv.at[pl.ds(0, DPS)])
                for i in range(DPS // 16):
                    sl = pl.ds(i * 16, 16)
                    dsum[sl] = dsum[sl] + degv[sl]
                return carry

            lax.fori_loop(0, NS, dred, 0)
            pltpu.sync_copy(dsum, deg_hbm.at[c, pl.ds(s * DPS, DPS)])

    return k(p_split, src_t, dst_t)


def _split_halves(p, out_ref):
    out_ref[:, 0, :] = p[:, 0:DH]
    out_ref[:, 1, :] = p[:, DH:D]


def _tc_project(x, Wl):
    def body(x_ref, wl_ref, out_ref):
        p = jnp.dot(x_ref[...], wl_ref[...], preferred_element_type=jnp.float32)
        _split_halves(p, out_ref)

    return pl.pallas_call(
        body,
        grid=(N // BLK,),
        in_specs=[
            pl.BlockSpec((BLK, D), lambda i: (i, 0)),
            pl.BlockSpec((D, D), lambda i: (0, 0)),
        ],
        out_specs=pl.BlockSpec((BLK, 2, DH), lambda i: (i, 0, 0)),
        out_shape=jax.ShapeDtypeStruct((N, 2, DH), jnp.float32),
    )(x, Wl)


def _agg_from_acc(acc_ref, d0_ref, d1_ref):
    deg = jnp.clip(d0_ref[...] + d1_ref[...], 1.0, None)
    agg = jnp.concatenate([acc_ref[0], acc_ref[1]], axis=1) / deg
    return agg


def _tc_mid(acc, d0, d1, x, Wr1, b1, Wl2, Wr2, b2):
    def body(acc_ref, d0_ref, d1_ref, x_ref, wr1_ref, b1_ref, wl2_ref,
             wr2_ref, b2_ref, p2_ref, r2_ref):
        agg = _agg_from_acc(acc_ref, d0_ref, d1_ref)
        h = jnp.maximum(
            agg + jnp.dot(x_ref[...], wr1_ref[...],
                          preferred_element_type=jnp.float32) + b1_ref[...],
            0.0)
        p2 = jnp.dot(h, wl2_ref[...], preferred_element_type=jnp.float32)
        _split_halves(p2, p2_ref)
        r2_ref[...] = jnp.dot(h, wr2_ref[...],
                              preferred_element_type=jnp.float32) + b2_ref[...]

    return pl.pallas_call(
        body,
        grid=(N // BLK,),
        in_specs=[
            pl.BlockSpec((NC, BLK, DH), lambda i: (0, i, 0)),
            pl.BlockSpec((BLK, 1), lambda i: (i, 0)),
            pl.BlockSpec((BLK, 1), lambda i: (i, 0)),
            pl.BlockSpec((BLK, D), lambda i: (i, 0)),
            pl.BlockSpec((D, D), lambda i: (0, 0)),
            pl.BlockSpec((1, D), lambda i: (0, 0)),
            pl.BlockSpec((D, D), lambda i: (0, 0)),
            pl.BlockSpec((D, D), lambda i: (0, 0)),
            pl.BlockSpec((1, D), lambda i: (0, 0)),
        ],
        out_specs=[
            pl.BlockSpec((BLK, 2, DH), lambda i: (i, 0, 0)),
            pl.BlockSpec((BLK, D), lambda i: (i, 0)),
        ],
        out_shape=[
            jax.ShapeDtypeStruct((N, 2, DH), jnp.float32),
            jax.ShapeDtypeStruct((N, D), jnp.float32),
        ],
    )(acc, d0, d1, x, Wr1, b1, Wl2, Wr2, b2)


def _tc_final(acc, d0, d1, r2):
    def body(acc_ref, d0_ref, d1_ref, r2_ref, out_ref):
        t = _agg_from_acc(acc_ref, d0_ref, d1_ref) + r2_ref[...]
        m = jnp.max(t, axis=-1, keepdims=True)
        lse = m + jnp.log(jnp.sum(jnp.exp(t - m), axis=-1, keepdims=True))
        out_ref[...] = t - lse

    return pl.pallas_call(
        body,
        grid=(N // BLK,),
        in_specs=[
            pl.BlockSpec((NC, BLK, DH), lambda i: (0, i, 0)),
            pl.BlockSpec((BLK, 1), lambda i: (i, 0)),
            pl.BlockSpec((BLK, 1), lambda i: (i, 0)),
            pl.BlockSpec((BLK, D), lambda i: (i, 0)),
        ],
        out_specs=pl.BlockSpec((BLK, D), lambda i: (i, 0)),
        out_shape=jax.ShapeDtypeStruct((N, D), jnp.float32),
    )(acc, d0, d1, r2)


def kernel(x, edge_index, Wl1, Wr1, b1, Wl2, Wr2, b2):
    src = edge_index[0].astype(jnp.int32).reshape(NS, NCH, G)
    src = jnp.stack([2 * src, 2 * src + 1])
    dst = edge_index[1].astype(jnp.int32).reshape(NS, NCH, G)
    b1r = b1.reshape(1, D)
    b2r = b2.reshape(1, D)

    p1 = _tc_project(x, Wl1).reshape(2 * N, DH)
    acc1, deg = _sc_segment_sum(p1, src, dst, compute_deg=True)
    d0 = deg[0, :N].reshape(N, 1)
    d1 = deg[1, :N].reshape(N, 1)
    p2, r2 = _tc_mid(acc1, d0, d1, x, Wr1, b1r, Wl2, Wr2, b2r)
    (acc2,) = _sc_segment_sum(p2.reshape(2 * N, DH), src, dst, compute_deg=False)
    return _tc_final(acc2, d0, d1, r2)
